# baseline (speedup 1.0000x reference)
import functools

import jax
import jax.numpy as jnp
from jax import lax
from jax.experimental import pallas as pl
from jax.experimental.pallas import tpu as pltpu
from jax.experimental.pallas import tpu_sc as plsc

_B, _L, _D, _V = 4096, 50, 64, 1000
_N = _B * _L
_NH = _N // 2
_NW = 32
_PER_W = _NH // _NW
_C = 128
_NCHUNK = _PER_W // _C
_EPS = 1e-5


def _lanesum16(x):
    dnums = lax.GatherDimensionNumbers(
        offset_dims=(), collapsed_slice_dims=(0,), start_index_map=(0,))
    for k in (1, 2, 4, 8):
        idx = lax.iota(jnp.int32, 16) ^ k
        x = x + lax.gather(x, idx[:, None], dnums, (1,),
                           mode=lax.GatherScatterMode.PROMISE_IN_BOUNDS)
    return x


def _rsqrt16(x):
    i = lax.bitcast_convert_type(x, jnp.int32)
    i = jnp.int32(0x5F3759DF) - lax.shift_right_logical(i, 1)
    y = lax.bitcast_convert_type(i, jnp.float32)
    half = x * 0.5
    for _ in range(2):
        y = y * (1.5 - half * y * y)
    return y


def _sc_body(bb, t1, t2, t3, t4, wb, out, raw_v, idx_v, r_v, ov, wbv,
             sem0, sem1):
    wid = lax.axis_index("s") * 2 + lax.axis_index("c")
    tabs = (t1, t2, t3, t4)
    sems = (sem0, sem1)

    pltpu.sync_copy(bb.at[pl.ds(wid * (_PER_W * 4), _PER_W * 4)], raw_v)
    pltpu.sync_copy(wb, wbv)

    wvec = [wbv[0, pl.ds(16 * q, 16)] for q in range(4)]
    bvec = [wbv[1, pl.ds(16 * q, 16)] for q in range(4)]

    lanes = lax.iota(jnp.int32, 16)

    def deinterleave(j, c):
        base = j * (4 * _C) + lanes * 4
        for k in range(4):
            for g in range(8):
                v = plsc.load_gather(raw_v, [base + (64 * g + k)])
                idx_v[c, k, pl.ds(16 * g, 16)] = v

    def fire(j, c):
        del j
        for k in range(4):
            pltpu.async_copy(
                tabs[k].at[idx_v.at[c, k]], r_v.at[c, k], sems[c])

    def drain(c):
        for k in range(4):
            pltpu.make_async_copy(
                tabs[k].at[idx_v.at[c, k]], r_v.at[c, k], sems[c]).wait()

    def compute(j, c):
        rv = [r_v.at[c, k] for k in range(4)]

        def token(t, carry2):
            s = []
            for q in range(4):
                sl = pl.ds(16 * q, 16)
                s.append(((rv[0][t, sl] + rv[1][t, sl])
                          + (rv[2][t, sl] + rv[3][t, sl])))
            tot = (s[0] + s[1]) + (s[2] + s[3])
            ssq = (s[0] * s[0] + s[1] * s[1]) + (s[2] * s[2] + s[3] * s[3])
            mu = _lanesum16(tot) * (1.0 / 64.0)
            var = _lanesum16(ssq) * (1.0 / 64.0) - mu * mu
            inv = _rsqrt16(var + _EPS)
            for q in range(4):
                ov[pl.ds(t * _D + 16 * q, 16)] = (
                    ((s[q] - mu) * inv) * wvec[q] + bvec[q])
            return carry2

        lax.fori_loop(0, _C, token, 0, unroll=8)
        pltpu.sync_copy(
            ov, out.at[pl.ds((wid * _PER_W + j * _C) * _D, _C * _D)])

    deinterleave(0, 0)
    fire(0, 0)

    def step(jj, carry):
        for b in (0, 1):
            j = 2 * jj + b
            jn = jnp.minimum(j + 1, _NCHUNK - 1)
            deinterleave(jn, 1 - b)
            fire(jn, 1 - b)
            drain(b)
            compute(j, b)
        return carry

    lax.fori_loop(0, _NCHUNK // 2, step, 0)
    drain(0)
    compute(_NCHUNK - 1, 0)


@functools.partial(jax.jit, static_argnums=())
def kernel(bbox, Wx1, Wy1, Wx2, Wy2, ln_w, ln_b):
    bb0 = bbox[:_B // 2].reshape(_NH * 4)
    bb1 = bbox[_B // 2:].reshape(_NH * 4)
    wb = jnp.stack([ln_w, ln_b])

    mesh = plsc.VectorSubcoreMesh(core_axis_name="c", subcore_axis_name="s")
    run = pl.kernel(
        _sc_body,
        mesh=mesh,
        compiler_params=pltpu.CompilerParams(
            use_tc_tiling_on_sc=False, needs_layout_passes=False),
        out_type=jax.ShapeDtypeStruct((_NH * _D,), jnp.float32),
        scratch_types=[
            pltpu.VMEM((_NCHUNK * 4 * _C,), jnp.int32),
            pltpu.VMEM((2, 4, _C), jnp.int32),
            pltpu.VMEM((2, 4, _C, _D), jnp.float32),
            pltpu.VMEM((_C * _D,), jnp.float32),
            pltpu.VMEM((2, _D), jnp.float32),
            pltpu.SemaphoreType.DMA,
            pltpu.SemaphoreType.DMA,
        ],
    )
    o0 = run(bb0, Wx1, Wy1, Wx2, Wy2, wb)
    o1 = run(bb1, Wx1, Wy1, Wx2, Wy2, wb)
    return jnp.concatenate([o0, o1]).reshape(_B, _L, _D)


# speedup vs baseline: 1.0192x; 1.0192x over previous
import functools

import jax
import jax.numpy as jnp
from jax import lax
from jax.experimental import pallas as pl
from jax.experimental.pallas import tpu as pltpu
from jax.experimental.pallas import tpu_sc as plsc

_B, _L, _D, _V = 4096, 50, 64, 1000
_N = _B * _L
_NH = _N // 2
_NW = 32
_PER_W = _NH // _NW
_C = 128
_NCHUNK = _PER_W // _C
_EPS = 1e-5


def _lanesum16(x):
    dnums = lax.GatherDimensionNumbers(
        offset_dims=(), collapsed_slice_dims=(0,), start_index_map=(0,))
    for k in (1, 2, 4, 8):
        idx = lax.iota(jnp.int32, 16) ^ k
        x = x + lax.gather(x, idx[:, None], dnums, (1,),
                           mode=lax.GatherScatterMode.PROMISE_IN_BOUNDS)
    return x


def _rsqrt16(x):
    i = lax.bitcast_convert_type(x, jnp.int32)
    i = jnp.int32(0x5F3759DF) - lax.shift_right_logical(i, 1)
    y = lax.bitcast_convert_type(i, jnp.float32)
    half = x * 0.5
    for _ in range(3):
        y = y * (1.5 - half * y * y)
    return y


def _sc_body(bb, t1, t2, t3, t4, wb, out, raw_v, idx_v, r_v, ov, wbv,
             sem0, sem1):
    wid = lax.axis_index("s") * 2 + lax.axis_index("c")
    tabs = (t1, t2, t3, t4)
    sems = (sem0, sem1)

    pltpu.sync_copy(bb.at[pl.ds(wid * (_PER_W * 4), _PER_W * 4)], raw_v)
    pltpu.sync_copy(wb, wbv)

    wvec = [wbv[0, pl.ds(16 * q, 16)] for q in range(4)]
    bvec = [wbv[1, pl.ds(16 * q, 16)] for q in range(4)]

    lanes = lax.iota(jnp.int32, 16)

    def deinterleave(j, c):
        base = j * (4 * _C) + lanes * 4
        for k in range(4):
            for g in range(8):
                v = plsc.load_gather(raw_v, [base + (64 * g + k)])
                idx_v[c, k, pl.ds(16 * g, 16)] = v

    def fire(j, c):
        del j
        for k in range(4):
            pltpu.async_copy(
                tabs[k].at[idx_v.at[c, k]], r_v.at[c, k], sems[c])

    def drain(c):
        for k in range(4):
            pltpu.make_async_copy(
                tabs[k].at[idx_v.at[c, k]], r_v.at[c, k], sems[c]).wait()

    def compute(j, c):
        rv = [r_v.at[c, k] for k in range(4)]

        def token(t, carry2):
            s = []
            for q in range(4):
                sl = pl.ds(16 * q, 16)
                s.append(((rv[0][t, sl] + rv[1][t, sl])
                          + (rv[2][t, sl] + rv[3][t, sl])))
            tot = (s[0] + s[1]) + (s[2] + s[3])
            ssq = (s[0] * s[0] + s[1] * s[1]) + (s[2] * s[2] + s[3] * s[3])
            mu = _lanesum16(tot) * (1.0 / 64.0)
            var = _lanesum16(ssq) * (1.0 / 64.0) - mu * mu
            inv = _rsqrt16(var + _EPS)
            for q in range(4):
                ov[pl.ds(t * _D + 16 * q, 16)] = (
                    ((s[q] - mu) * inv) * wvec[q] + bvec[q])
            return carry2

        lax.fori_loop(0, _C, token, 0, unroll=4)
        pltpu.sync_copy(
            ov, out.at[pl.ds((wid * _PER_W + j * _C) * _D, _C * _D)])

    deinterleave(0, 0)
    fire(0, 0)

    def step(jj, carry):
        for b in (0, 1):
            j = 2 * jj + b
            jn = jnp.minimum(j + 1, _NCHUNK - 1)
            deinterleave(jn, 1 - b)
            fire(jn, 1 - b)
            drain(b)
            compute(j, b)
        return carry

    lax.fori_loop(0, _NCHUNK // 2, step, 0)
    drain(0)
    compute(_NCHUNK - 1, 0)


@functools.partial(jax.jit, static_argnums=())
def kernel(bbox, Wx1, Wy1, Wx2, Wy2, ln_w, ln_b):
    bb0 = bbox[:_B // 2].reshape(_NH * 4)
    bb1 = bbox[_B // 2:].reshape(_NH * 4)
    wb = jnp.stack([ln_w, ln_b])

    mesh = plsc.VectorSubcoreMesh(core_axis_name="c", subcore_axis_name="s")
    run = pl.kernel(
        _sc_body,
        mesh=mesh,
        compiler_params=pltpu.CompilerParams(
            use_tc_tiling_on_sc=False, needs_layout_passes=False),
        out_type=jax.ShapeDtypeStruct((_NH * _D,), jnp.float32),
        scratch_types=[
            pltpu.VMEM((_NCHUNK * 4 * _C,), jnp.int32),
            pltpu.VMEM((2, 4, _C), jnp.int32),
            pltpu.VMEM((2, 4, _C, _D), jnp.float32),
            pltpu.VMEM((_C * _D,), jnp.float32),
            pltpu.VMEM((2, _D), jnp.float32),
            pltpu.SemaphoreType.DMA,
            pltpu.SemaphoreType.DMA,
        ],
    )
    o0 = run(bb0, Wx1, Wy1, Wx2, Wy2, wb)
    o1 = run(bb1, Wx1, Wy1, Wx2, Wy2, wb)
    return jnp.concatenate([o0, o1]).reshape(_B, _L, _D)
